# baseline (speedup 1.0000x reference)
import functools

import jax
import jax.numpy as jnp
from jax import lax
from jax.experimental import pallas as pl
from jax.experimental.pallas import tpu as pltpu
from jax.experimental.pallas import tpu_sc as plsc

B = 16384
D = 32
N = 1000000
PACK = 128 // D
NP = N // PACK
NC = 2
NS = 16
NW = NC * NS
BPW = B // NW


def _sc_gather(users_p, items_p, uidx, iidx):
  mesh = plsc.VectorSubcoreMesh(core_axis_name="c", subcore_axis_name="s")

  @functools.partial(
      pl.kernel,
      mesh=mesh,
      out_type=[
          jax.ShapeDtypeStruct((B, 128), jnp.float32),
          jax.ShapeDtypeStruct((B, 128), jnp.float32),
      ],
      scratch_types=[
          pltpu.VMEM((4, 128), jnp.int32),
          pltpu.VMEM((4, 128), jnp.int32),
          pltpu.VMEM((128, 128), jnp.float32),
          pltpu.VMEM((128, 128), jnp.float32),
          pltpu.SemaphoreType.DMA,
          pltpu.SemaphoreType.DMA,
      ],
  )
  def k(users_hbm, items_hbm, uid_hbm, iid_hbm, u_out, v_out,
        uid_v, iid_v, urows_v, vrows_v, sem_u, sem_v):
    wid = lax.axis_index("s") * NC + lax.axis_index("c")
    base = wid * BPW
    for h in range(4):
      pltpu.sync_copy(uid_hbm.at[pl.ds(base + h * 128, 128)], uid_v.at[h])
      pltpu.sync_copy(iid_hbm.at[pl.ds(base + h * 128, 128)], iid_v.at[h])
    for h in range(4):
      cu = pltpu.async_copy(users_hbm.at[uid_v.at[h]], urows_v, sem_u)
      cv = pltpu.async_copy(items_hbm.at[iid_v.at[h]], vrows_v, sem_v)
      cu.wait()
      cv.wait()
      pltpu.sync_copy(urows_v, u_out.at[pl.ds(base + h * 128, 128)])
      pltpu.sync_copy(vrows_v, v_out.at[pl.ds(base + h * 128, 128)])

  return k(users_p, items_p, uidx, iidx)


TBLK = 512


def _tc_pack(table_t):
  def body(x_ref, o_ref):
    parts = [jnp.transpose(x_ref[:, m * 128:(m + 1) * 128])
             for m in range(PACK)]
    o_ref[...] = jnp.concatenate(parts, axis=1)

  return pl.pallas_call(
      body,
      grid=((N + TBLK - 1) // TBLK,),
      in_specs=[pl.BlockSpec((32, TBLK), lambda i: (0, i))],
      out_shape=jax.ShapeDtypeStruct((NP, 128), jnp.float32),
      out_specs=pl.BlockSpec((TBLK // PACK, 128), lambda i: (i, 0)),
      compiler_params=pltpu.CompilerParams(
          dimension_semantics=("arbitrary",),
      ),
  )(table_t)


GB = 2048


def _tc_loss(u_big, v_big, user_id, item_id, rating):
  def body(u_ref, v_ref, uid_ref, iid_ref, r_ref, o_ref):
    i = pl.program_id(0)

    @pl.when(i == 0)
    def _():
      o_ref[0, 0] = 0.0

    uoff = (uid_ref[...] >> 7) & (PACK - 1)
    ioff = (iid_ref[...] >> 7) & (PACK - 1)
    pred = jnp.zeros((GB,), jnp.float32)
    for a in range(PACK):
      ua = u_ref[:, a * D:(a + 1) * D]
      for b in range(PACK):
        vb = v_ref[:, b * D:(b + 1) * D]
        d = jnp.sum(ua * vb, axis=1)
        m = jnp.logical_and(uoff == a, ioff == b)
        pred = pred + jnp.where(m, d, 0.0)
    err = r_ref[...] - pred
    o_ref[0, 0] += jnp.sum(err * err) * (1.0 / B)

  return pl.pallas_call(
      body,
      grid=(B // GB,),
      in_specs=[
          pl.BlockSpec((GB, 128), lambda i: (i, 0)),
          pl.BlockSpec((GB, 128), lambda i: (i, 0)),
          pl.BlockSpec((GB,), lambda i: (i,)),
          pl.BlockSpec((GB,), lambda i: (i,)),
          pl.BlockSpec((GB,), lambda i: (i,)),
      ],
      out_shape=jax.ShapeDtypeStruct((1, 1), jnp.float32),
      out_specs=pl.BlockSpec(memory_space=pltpu.SMEM),
  )(u_big, v_big, user_id, item_id, rating)


def kernel(user_id, item_id, rating, users, items):
  users_p = _tc_pack(users.T)
  items_p = _tc_pack(items.T)
  uidx = ((user_id >> 9) << 7) + (user_id & 127)
  iidx = ((item_id >> 9) << 7) + (item_id & 127)
  u_big, v_big = _sc_gather(users_p, items_p, uidx, iidx)
  return _tc_loss(u_big, v_big, user_id, item_id, rating)[0, 0]


# speedup vs baseline: 3.2512x; 3.2512x over previous
import functools

import jax
import jax.numpy as jnp
from jax import lax
from jax.experimental import pallas as pl
from jax.experimental.pallas import tpu as pltpu
from jax.experimental.pallas import tpu_sc as plsc

B = 16384
D = 32
N = 1000000
PACK = 128 // D
NP = N // PACK
NC = 2
NS = 16
NW = NC * NS
BPW = B // NW


def _sc_gather(users_p, items_p, uidx, iidx):
  mesh = plsc.VectorSubcoreMesh(core_axis_name="c", subcore_axis_name="s")

  @functools.partial(
      pl.kernel,
      mesh=mesh,
      out_type=[
          jax.ShapeDtypeStruct((B, 128), jnp.float32),
          jax.ShapeDtypeStruct((B, 128), jnp.float32),
      ],
      scratch_types=[
          pltpu.VMEM((4, 128), jnp.int32),
          pltpu.VMEM((4, 128), jnp.int32),
          pltpu.VMEM((128, 128), jnp.float32),
          pltpu.VMEM((128, 128), jnp.float32),
          pltpu.SemaphoreType.DMA,
          pltpu.SemaphoreType.DMA,
      ],
  )
  def k(users_hbm, items_hbm, uid_hbm, iid_hbm, u_out, v_out,
        uid_v, iid_v, urows_v, vrows_v, sem_u, sem_v):
    wid = lax.axis_index("s") * NC + lax.axis_index("c")
    base = wid * BPW
    for h in range(4):
      pltpu.sync_copy(uid_hbm.at[pl.ds(base + h * 128, 128)], uid_v.at[h])
      pltpu.sync_copy(iid_hbm.at[pl.ds(base + h * 128, 128)], iid_v.at[h])
    for h in range(4):
      cu = pltpu.async_copy(users_hbm.at[uid_v.at[h]], urows_v, sem_u)
      cv = pltpu.async_copy(items_hbm.at[iid_v.at[h]], vrows_v, sem_v)
      cu.wait()
      cv.wait()
      pltpu.sync_copy(urows_v, u_out.at[pl.ds(base + h * 128, 128)])
      pltpu.sync_copy(vrows_v, v_out.at[pl.ds(base + h * 128, 128)])

  return k(users_p, items_p, uidx, iidx)


TBLK = 4096


def _tc_pack(table_t):
  def body(x_ref, o_ref):
    for g in range(TBLK // (PACK * 128)):
      parts = [jnp.transpose(x_ref[:, g * 512 + m * 128:g * 512 + (m + 1) * 128])
               for m in range(PACK)]
      o_ref[g * 128:(g + 1) * 128, :] = jnp.concatenate(parts, axis=1)

  return pl.pallas_call(
      body,
      grid=((N + TBLK - 1) // TBLK,),
      in_specs=[pl.BlockSpec((32, TBLK), lambda i: (0, i))],
      out_shape=jax.ShapeDtypeStruct((NP, 128), jnp.float32),
      out_specs=pl.BlockSpec((TBLK // PACK, 128), lambda i: (i, 0)),
      compiler_params=pltpu.CompilerParams(
          dimension_semantics=("parallel",),
      ),
  )(table_t)


GB = 2048


def _tc_loss(u_big, v_big, user_id, item_id, rating):
  def body(u_ref, v_ref, uid_ref, iid_ref, r_ref, o_ref):
    i = pl.program_id(0)

    @pl.when(i == 0)
    def _():
      o_ref[0, 0] = 0.0

    uoff = (uid_ref[...] >> 7) & (PACK - 1)
    ioff = (iid_ref[...] >> 7) & (PACK - 1)
    pred = jnp.zeros((GB,), jnp.float32)
    for a in range(PACK):
      ua = u_ref[:, a * D:(a + 1) * D]
      for b in range(PACK):
        vb = v_ref[:, b * D:(b + 1) * D]
        d = jnp.sum(ua * vb, axis=1)
        m = jnp.logical_and(uoff == a, ioff == b)
        pred = pred + jnp.where(m, d, 0.0)
    err = r_ref[...] - pred
    o_ref[0, 0] += jnp.sum(err * err) * (1.0 / B)

  return pl.pallas_call(
      body,
      grid=(B // GB,),
      in_specs=[
          pl.BlockSpec((GB, 128), lambda i: (i, 0)),
          pl.BlockSpec((GB, 128), lambda i: (i, 0)),
          pl.BlockSpec((GB,), lambda i: (i,)),
          pl.BlockSpec((GB,), lambda i: (i,)),
          pl.BlockSpec((GB,), lambda i: (i,)),
      ],
      out_shape=jax.ShapeDtypeStruct((1, 1), jnp.float32),
      out_specs=pl.BlockSpec(memory_space=pltpu.SMEM),
  )(u_big, v_big, user_id, item_id, rating)


def kernel(user_id, item_id, rating, users, items):
  users_p = _tc_pack(users.T)
  items_p = _tc_pack(items.T)
  uidx = ((user_id >> 9) << 7) + (user_id & 127)
  iidx = ((item_id >> 9) << 7) + (item_id & 127)
  u_big, v_big = _sc_gather(users_p, items_p, uidx, iidx)
  return _tc_loss(u_big, v_big, user_id, item_id, rating)[0, 0]


# speedup vs baseline: 3.7739x; 1.1608x over previous
import functools

import jax
import jax.numpy as jnp
from jax import lax
from jax.experimental import pallas as pl
from jax.experimental.pallas import tpu as pltpu
from jax.experimental.pallas import tpu_sc as plsc

B = 16384
D = 32
N = 1000000
PACK = 128 // D
NP = N // PACK
NC = 2
NS = 16
NW = NC * NS
BPW = B // NW


def _sc_gather(users_p, items_p, uidx, iidx):
  mesh = plsc.VectorSubcoreMesh(core_axis_name="c", subcore_axis_name="s")

  @functools.partial(
      pl.kernel,
      mesh=mesh,
      out_type=[
          jax.ShapeDtypeStruct((B, 128), jnp.float32),
          jax.ShapeDtypeStruct((B, 128), jnp.float32),
      ],
      scratch_types=[
          pltpu.VMEM((4, 128), jnp.int32),
          pltpu.VMEM((4, 128), jnp.int32),
          pltpu.VMEM((128, 128), jnp.float32),
          pltpu.VMEM((128, 128), jnp.float32),
          pltpu.SemaphoreType.DMA,
          pltpu.SemaphoreType.DMA,
      ],
  )
  def k(users_hbm, items_hbm, uid_hbm, iid_hbm, u_out, v_out,
        uid_v, iid_v, urows_v, vrows_v, sem_u, sem_v):
    wid = lax.axis_index("s") * NC + lax.axis_index("c")
    base = wid * BPW
    for h in range(4):
      pltpu.sync_copy(uid_hbm.at[pl.ds(base + h * 128, 128)], uid_v.at[h])
      pltpu.sync_copy(iid_hbm.at[pl.ds(base + h * 128, 128)], iid_v.at[h])
    for h in range(4):
      cu = pltpu.async_copy(users_hbm.at[uid_v.at[h]], urows_v, sem_u)
      cv = pltpu.async_copy(items_hbm.at[iid_v.at[h]], vrows_v, sem_v)
      cu.wait()
      cv.wait()
      pltpu.sync_copy(urows_v, u_out.at[pl.ds(base + h * 128, 128)])
      pltpu.sync_copy(vrows_v, v_out.at[pl.ds(base + h * 128, 128)])

  return k(users_p, items_p, uidx, iidx)


TBLK = 8192


def _tc_pack(table_t):
  def body(x_ref, o_ref):
    x = x_ref[...]
    for g in range(TBLK // (PACK * 128)):
      parts = [jnp.transpose(x[:, g * 512 + m * 128:g * 512 + (m + 1) * 128])
               for m in range(PACK)]
      o_ref[g * 128:(g + 1) * 128, :] = jnp.concatenate(parts, axis=1)

  return pl.pallas_call(
      body,
      grid=((N + TBLK - 1) // TBLK,),
      in_specs=[pl.BlockSpec((32, TBLK), lambda i: (0, i))],
      out_shape=jax.ShapeDtypeStruct((NP, 128), jnp.float32),
      out_specs=pl.BlockSpec((TBLK // PACK, 128), lambda i: (i, 0)),
      compiler_params=pltpu.CompilerParams(
          dimension_semantics=("parallel",),
      ),
  )(table_t)


GB = 2048


def _tc_loss(u_big, v_big, user_id, item_id, rating):
  def body(u_ref, v_ref, uid_ref, iid_ref, r_ref, o_ref):
    i = pl.program_id(0)

    @pl.when(i == 0)
    def _():
      o_ref[0, 0] = 0.0

    uoff = (uid_ref[...] >> 7) & (PACK - 1)
    ioff = (iid_ref[...] >> 7) & (PACK - 1)
    uf = u_ref[...].astype(jnp.float32)
    vf = v_ref[...].astype(jnp.float32)
    pred = jnp.zeros((GB,), jnp.float32)
    for a in range(PACK):
      ua = uf[:, a * D:(a + 1) * D]
      for b in range(PACK):
        vb = vf[:, b * D:(b + 1) * D]
        d = jnp.sum(ua * vb, axis=1)
        m = jnp.logical_and(uoff == a, ioff == b)
        pred = pred + jnp.where(m, d, 0.0)
    err = r_ref[...] - pred
    o_ref[0, 0] += jnp.sum(err * err) * (1.0 / B)

  return pl.pallas_call(
      body,
      grid=(B // GB,),
      in_specs=[
          pl.BlockSpec((GB, 128), lambda i: (i, 0)),
          pl.BlockSpec((GB, 128), lambda i: (i, 0)),
          pl.BlockSpec((GB,), lambda i: (i,)),
          pl.BlockSpec((GB,), lambda i: (i,)),
          pl.BlockSpec((GB,), lambda i: (i,)),
      ],
      out_shape=jax.ShapeDtypeStruct((1, 1), jnp.float32),
      out_specs=pl.BlockSpec(memory_space=pltpu.SMEM),
  )(u_big, v_big, user_id, item_id, rating)


def kernel(user_id, item_id, rating, users, items):
  users_p = _tc_pack(users.T)
  items_p = _tc_pack(items.T)
  uidx = ((user_id >> 9) << 7) + (user_id & 127)
  iidx = ((item_id >> 9) << 7) + (item_id & 127)
  u_big, v_big = _sc_gather(users_p, items_p, uidx, iidx)
  return _tc_loss(u_big, v_big, user_id, item_id, rating)[0, 0]
